# baseline (speedup 1.0000x reference)
import functools

import jax
import jax.numpy as jnp
from jax import lax
from jax.experimental import pallas as pl
from jax.experimental.pallas import tpu as pltpu
from jax.experimental.pallas import tpu_sc as plsc

NC, NS, L = 2, 16, 16
NW = NC * NS
B, S, D, H, C = 4096, 200, 64, 128, 4
BPW = B // NW
CH = 100
NCH = S // CH
KD = D // L
ROWS_PER_ITER = 8
NPHASE = 4


def _pool_sc_kernel(idx_hbm, table_hbm, out_hbm, idx_v, buf_a, buf_b, out_v,
                    sem_a, sem_b):
    wid = lax.axis_index("s") * NC + lax.axis_index("c")

    pltpu.sync_copy(idx_hbm.at[wid], idx_v)

    def issue(row, buf, sem):
        for h in range(NCH):
            pltpu.async_copy(table_hbm.at[idx_v.at[row * NCH + h]],
                             buf.at[pl.ds(h * CH, CH)], sem)

    def drain(row, buf, sem):
        for h in range(NCH):
            pltpu.make_async_copy(table_hbm.at[idx_v.at[row * NCH + h]],
                                  buf.at[pl.ds(h * CH, CH)], sem).wait()

    def accumulate(row, buf):
        zero = jnp.zeros((L,), jnp.float32)
        init = tuple(zero for _ in range(NPHASE * KD))

        def body(i, accs):
            accs = list(accs)
            base = i * ROWS_PER_ITER
            for u in range(ROWS_PER_ITER):
                p = u % NPHASE
                for k in range(KD):
                    v = buf[base + u, pl.ds(k * L, L)]
                    accs[p * KD + k] = accs[p * KD + k] + v
            return tuple(accs)

        accs = lax.fori_loop(0, S // ROWS_PER_ITER, body, init)
        for k in range(KD):
            tot = accs[k]
            for p in range(1, NPHASE):
                tot = tot + accs[p * KD + k]
            out_v[row, pl.ds(k * L, L)] = tot

    issue(0, buf_a, sem_a)

    def row_pair(rr, _):
        r0 = rr * 2
        issue(r0 + 1, buf_b, sem_b)
        drain(r0, buf_a, sem_a)
        accumulate(r0, buf_a)

        @pl.when(r0 + 2 < BPW)
        def _():
            issue(r0 + 2, buf_a, sem_a)

        drain(r0 + 1, buf_b, sem_b)
        accumulate(r0 + 1, buf_b)
        return 0

    lax.fori_loop(0, BPW // 2, row_pair, 0)

    pltpu.sync_copy(out_v, out_hbm.at[pl.ds(wid * BPW, BPW)])


@functools.partial(
    pl.kernel,
    out_type=jax.ShapeDtypeStruct((B, D), jnp.float32),
    mesh=plsc.VectorSubcoreMesh(core_axis_name="c", subcore_axis_name="s",
                                num_cores=NC, num_subcores=NS),
    scratch_types=[
        pltpu.VMEM((BPW * NCH, CH), jnp.int32),
        pltpu.VMEM((S, D), jnp.float32),
        pltpu.VMEM((S, D), jnp.float32),
        pltpu.VMEM((BPW, D), jnp.float32),
        pltpu.SemaphoreType.DMA,
        pltpu.SemaphoreType.DMA,
    ],
    compiler_params=pltpu.CompilerParams(use_tc_tiling_on_sc=False),
)
def _pool_sc(idx_hbm, table_hbm, out_hbm, idx_v, buf_a, buf_b, out_v,
             sem_a, sem_b):
    _pool_sc_kernel(idx_hbm, table_hbm, out_hbm, idx_v, buf_a, buf_b, out_v,
                    sem_a, sem_b)


def _mlp_body(x_ref, w1_ref, b1_ref, w2_ref, b2_ref, o_ref):
    x = x_ref[...]
    h = jnp.dot(x, w1_ref[...], preferred_element_type=jnp.float32)
    h = jnp.maximum(h + b1_ref[...], 0.0)
    o = jnp.dot(h, w2_ref[...], preferred_element_type=jnp.float32)
    o_ref[...] = o + b2_ref[...]


def _mlp_tc(x, w1_scaled, b1, w2, b2):
    return pl.pallas_call(
        _mlp_body,
        out_shape=jax.ShapeDtypeStruct((B, C), jnp.float32),
    )(x, w1_scaled, b1.reshape(1, H), w2, b2.reshape(1, C))


def kernel(reports, table, W1, b1, W2, b2):
    idx3 = reports.reshape(NW, BPW * NCH, CH)
    pooled_sums = _pool_sc(idx3, table)
    return _mlp_tc(pooled_sums, W1 * (1.0 / S), b1, W2, b2)


# speedup vs baseline: 1.0026x; 1.0026x over previous
import functools

import jax
import jax.numpy as jnp
from jax import lax
from jax.experimental import pallas as pl
from jax.experimental.pallas import tpu as pltpu
from jax.experimental.pallas import tpu_sc as plsc

NC, NS, L = 2, 16, 16
NW = NC * NS
B, S, D, H, C = 4096, 200, 64, 128, 4
BPW = B // NW
CHUNKS = ((0, 104), (104, 96))
KD = D // L
ROWS_PER_ITER = 8
NPHASE = 4


def _pool_sc_kernel(idx_hbm, table_hbm, out_hbm, idx_v, buf_a, buf_b, out_v,
                    sem_a, sem_b):
    wid = lax.axis_index("s") * NC + lax.axis_index("c")

    pltpu.sync_copy(idx_hbm.at[pl.ds(wid * BPW, BPW)], idx_v)

    def issue(row, buf, sem):
        for off, sz in CHUNKS:
            pltpu.async_copy(table_hbm.at[idx_v.at[row, pl.ds(off, sz)]],
                             buf.at[pl.ds(off, sz)], sem)

    def drain(row, buf, sem):
        for off, sz in CHUNKS:
            pltpu.make_async_copy(table_hbm.at[idx_v.at[row, pl.ds(off, sz)]],
                                  buf.at[pl.ds(off, sz)], sem).wait()

    def accumulate(row, buf):
        zero = jnp.zeros((L,), jnp.float32)
        init = tuple(zero for _ in range(NPHASE * KD))

        def body(i, accs):
            accs = list(accs)
            base = i * ROWS_PER_ITER
            for u in range(ROWS_PER_ITER):
                p = u % NPHASE
                for k in range(KD):
                    v = buf[base + u, pl.ds(k * L, L)]
                    accs[p * KD + k] = accs[p * KD + k] + v
            return tuple(accs)

        accs = lax.fori_loop(0, S // ROWS_PER_ITER, body, init)
        for k in range(KD):
            tot = accs[k]
            for p in range(1, NPHASE):
                tot = tot + accs[p * KD + k]
            out_v[row, pl.ds(k * L, L)] = tot

    issue(0, buf_a, sem_a)

    def row_pair(rr, _):
        r0 = rr * 2
        issue(r0 + 1, buf_b, sem_b)
        drain(r0, buf_a, sem_a)
        accumulate(r0, buf_a)

        @pl.when(r0 + 2 < BPW)
        def _():
            issue(r0 + 2, buf_a, sem_a)

        drain(r0 + 1, buf_b, sem_b)
        accumulate(r0 + 1, buf_b)
        return 0

    lax.fori_loop(0, BPW // 2, row_pair, 0)

    pltpu.sync_copy(out_v, out_hbm.at[pl.ds(wid * BPW, BPW)])


@functools.partial(
    pl.kernel,
    out_type=jax.ShapeDtypeStruct((B, D), jnp.float32),
    mesh=plsc.VectorSubcoreMesh(core_axis_name="c", subcore_axis_name="s",
                                num_cores=NC, num_subcores=NS),
    scratch_types=[
        pltpu.VMEM((BPW, S), jnp.int32),
        pltpu.VMEM((S, D), jnp.float32),
        pltpu.VMEM((S, D), jnp.float32),
        pltpu.VMEM((BPW, D), jnp.float32),
        pltpu.SemaphoreType.DMA,
        pltpu.SemaphoreType.DMA,
    ],
    compiler_params=pltpu.CompilerParams(use_tc_tiling_on_sc=False),
)
def _pool_sc(idx_hbm, table_hbm, out_hbm, idx_v, buf_a, buf_b, out_v,
             sem_a, sem_b):
    _pool_sc_kernel(idx_hbm, table_hbm, out_hbm, idx_v, buf_a, buf_b, out_v,
                    sem_a, sem_b)


def _mlp_body(x_ref, w1_ref, b1_ref, w2_ref, b2_ref, o_ref):
    x = x_ref[...]
    h = jnp.dot(x, w1_ref[...], preferred_element_type=jnp.float32)
    h = jnp.maximum(h + b1_ref[...], 0.0)
    o = jnp.dot(h, w2_ref[...], preferred_element_type=jnp.float32)
    o_ref[...] = o + b2_ref[...]


def _mlp_tc(x, w1_scaled, b1, w2, b2):
    return pl.pallas_call(
        _mlp_body,
        out_shape=jax.ShapeDtypeStruct((B, C), jnp.float32),
    )(x, w1_scaled, b1.reshape(1, H), w2, b2.reshape(1, C))


def kernel(reports, table, W1, b1, W2, b2):
    pooled_sums = _pool_sc(reports, table)
    return _mlp_tc(pooled_sums, W1 * (1.0 / S), b1, W2, b2)
